# Initial kernel scaffold; baseline (speedup 1.0000x reference)
import functools

import jax
import jax.numpy as jnp
from jax import lax
from jax.experimental import pallas as pl
from jax.experimental.pallas import tpu as pltpu
from jax.experimental.pallas import tpu_sc as plsc

N = 10000
E = 320000
NG = 64
EPS = 0.1
GAMMA = 0.1

NTILE = 16
NP = 10240
TRASH = 10000
EPT = E // NTILE
KC = 80
NCHUNK = EPT // KC
RPT = NP // NTILE
ZR = 64


def _sc_conv(dh):
    mesh = plsc.VectorSubcoreMesh(core_axis_name="c", subcore_axis_name="s")
    out = jax.ShapeDtypeStruct((NP, dh), jnp.float32)

    @functools.partial(
        pl.kernel,
        out_type=[out] * 4,
        mesh=mesh,
        scratch_types=[
            pltpu.VMEM((NCHUNK, KC), jnp.int32),
            pltpu.VMEM((NCHUNK, KC), jnp.int32),
            pltpu.VMEM((KC, dh), jnp.float32),
            pltpu.VMEM((ZR, dh), jnp.float32),
            pltpu.VMEM_SHARED((NP, dh), jnp.float32),
            pltpu.SemaphoreType.DMA,
        ],
    )
    def conv(src3, dsth3, dstt3, yh0, yh1, yt0, yt1,
             ah0, ah1, at0, at1,
             src_v, dst_v, rows_v, zbuf, shared, sem):
        c = lax.axis_index("c")
        s = lax.axis_index("s")
        z16 = jnp.zeros((16,), jnp.float32)

        def zb(j, carry):
            for k in range(dh // 16):
                zbuf[j, pl.ds(16 * k, 16)] = z16
            return carry

        lax.fori_loop(0, ZR, zb, 0)
        pltpu.sync_copy(src3.at[s], src_v)

        def run_half(dst3, y0, y1, o0, o1):
            pltpu.sync_copy(dst3.at[s], dst_v)

            def zr(j, carry):
                pltpu.sync_copy(zbuf, shared.at[pl.ds(s * RPT + j * ZR, ZR)])
                return carry

            lax.fori_loop(0, RPT // ZR, zr, 0)
            plsc.subcore_barrier()

            def scat(y):
                def body(i, carry):
                    pltpu.async_copy(y.at[src_v.at[i]], rows_v, sem).wait()
                    pltpu.sync_copy(rows_v, shared.at[dst_v.at[i]], add=True)
                    return carry

                lax.fori_loop(0, NCHUNK, body, 0)

            pl.when(c == 0)(lambda: scat(y0))
            pl.when(c == 1)(lambda: scat(y1))
            plsc.subcore_barrier()

            def cp(o):
                pltpu.sync_copy(shared.at[pl.ds(s * RPT, RPT)],
                                o.at[pl.ds(s * RPT, RPT)])

            pl.when(c == 0)(lambda: cp(o0))
            pl.when(c == 1)(lambda: cp(o1))
            plsc.subcore_barrier()

        run_half(dsth3, yh0, yh1, ah0, ah1)
        run_half(dstt3, yt0, yt1, at0, at1)

    return conv


def _sc_readout():
    mesh = plsc.VectorSubcoreMesh(core_axis_name="c", subcore_axis_name="s")

    @functools.partial(
        pl.kernel,
        out_type=jax.ShapeDtypeStruct((NG, 512), jnp.float32),
        mesh=mesh,
        scratch_types=[
            pltpu.VMEM((72,), jnp.int32),
            pltpu.VMEM((8, 256), jnp.float32),
            pltpu.VMEM((1, 512), jnp.float32),
        ],
    )
    def ro(h_hbm, starts_hbm, out_hbm, starts_v, buf, obuf):
        c = lax.axis_index("c")
        s = lax.axis_index("s")
        wid = s * 2 + c
        pltpu.sync_copy(starts_hbm, starts_v)
        neg = jnp.float32(-3e38)

        for gg in range(2):
            g = wid * 2 + gg
            rs = starts_v[g]
            re = starts_v[g + 1]
            n = re - rs
            nch = (n + 7) // 8
            init = (tuple(jnp.zeros((16,), jnp.float32) for _ in range(16)),
                    tuple(jnp.full((16,), neg) for _ in range(16)))

            def chunk(ci, carry, rs=rs, re=re):
                sa, ma = carry
                base = rs + ci * 8
                st = jnp.minimum(base, N - 8)
                pltpu.sync_copy(h_hbm.at[pl.ds(st, 8)], buf)
                sa = list(sa)
                ma = list(ma)
                for r in range(8):
                    rid = st + r
                    valid = jnp.logical_and(rid >= base, rid < re)
                    vb = valid.astype(jnp.float32)
                    nvb = (jnp.float32(1.0) - vb) * neg
                    for k in range(16):
                        v = buf[r, pl.ds(16 * k, 16)]
                        sa[k] = sa[k] + v * vb
                        ma[k] = jnp.maximum(ma[k], v * vb + nvb)
                return (tuple(sa), tuple(ma))

            sa, ma = lax.fori_loop(0, nch, chunk, init)
            inv = jnp.float32(1.0) / jnp.maximum(n.astype(jnp.float32), 1.0)
            pos = (n > 0).astype(jnp.float32)
            for k in range(16):
                obuf[0, pl.ds(16 * k, 16)] = ma[k] * pos
                obuf[0, pl.ds(256 + 16 * k, 16)] = sa[k] * inv
            pltpu.sync_copy(obuf, out_hbm.at[pl.ds(g, 1)])

    return ro


_BN = 400
_GN = N // _BN


def _mm(a, b):
    return jnp.dot(a, b, preferred_element_type=jnp.float32)


def _tc_a0(x, pre_W, pre_b, Ph, Pt):
    def body(x_r, w_r, b_r, ph_r, pt_r, h_r, yh0_r, yh1_r, yt0_r, yt1_r):
        h = jnp.maximum(_mm(x_r[...], w_r[...]) + b_r[...], 0.0)
        h_r[...] = h
        yh = _mm(h, ph_r[...])
        yt = _mm(h, pt_r[...])
        yh0_r[...] = yh[:, :64]
        yh1_r[...] = yh[:, 64:]
        yt0_r[...] = yt[:, :64]
        yt1_r[...] = yt[:, 64:]

    full = lambda i: (0, 0)
    blk = lambda i: (i, 0)
    return pl.pallas_call(
        body,
        grid=(_GN,),
        in_specs=[pl.BlockSpec((_BN, 128), blk),
                  pl.BlockSpec((128, 128), full),
                  pl.BlockSpec((1, 128), full),
                  pl.BlockSpec((128, 128), full),
                  pl.BlockSpec((128, 128), full)],
        out_specs=[pl.BlockSpec((_BN, 128), blk)]
        + [pl.BlockSpec((_BN, 64), blk)] * 4,
        out_shape=[jax.ShapeDtypeStruct((N, 128), jnp.float32)]
        + [jax.ShapeDtypeStruct((N, 64), jnp.float32)] * 4,
    )(x, pre_W, pre_b, Ph, Pt)


def _tc_a(h, Ph, Pt):
    def body(h_r, ph_r, pt_r, yh0_r, yh1_r, yt0_r, yt1_r):
        hh = h_r[...]
        yh = _mm(hh, ph_r[...])
        yt = _mm(hh, pt_r[...])
        yh0_r[...] = yh[:, :128]
        yh1_r[...] = yh[:, 128:]
        yt0_r[...] = yt[:, :128]
        yt1_r[...] = yt[:, 128:]

    full = lambda i: (0, 0)
    blk = lambda i: (i, 0)
    return pl.pallas_call(
        body,
        grid=(_GN,),
        in_specs=[pl.BlockSpec((_BN, 256), blk),
                  pl.BlockSpec((256, 256), full),
                  pl.BlockSpec((256, 256), full)],
        out_specs=[pl.BlockSpec((_BN, 128), blk)] * 4,
        out_shape=[jax.ShapeDtypeStruct((N, 128), jnp.float32)] * 4,
    )(h, Ph, Pt)


def _tc_b(h, ah0, ah1, at0, at1, Ah, At, bh, bt, d, concat):
    dh = d // 2

    def body(h_r, ah0_r, ah1_r, at0_r, at1_r, Ah_r, At_r, bh_r, bt_r, o_r):
        hh = h_r[...]
        aggh = jnp.concatenate([ah0_r[...], ah1_r[...]], axis=1)
        aggt = jnp.concatenate([at0_r[...], at1_r[...]], axis=1)
        xh = jnp.maximum(
            hh + EPS * jnp.tanh(_mm(hh, Ah_r[...]) + aggh + bh_r[...]), 0.0)
        xt = jnp.maximum(
            hh + EPS * jnp.tanh(_mm(hh, At_r[...]) + aggt + bt_r[...]), 0.0)
        if concat:
            o_r[...] = jnp.concatenate([xh, xt], axis=1)
        else:
            o_r[...] = xh + xt

    full = lambda i: (0, 0)
    blk = lambda i: (i, 0)
    return pl.pallas_call(
        body,
        grid=(_GN,),
        in_specs=[pl.BlockSpec((_BN, d), blk)]
        + [pl.BlockSpec((_BN, dh), blk)] * 4
        + [pl.BlockSpec((d, d), full)] * 2
        + [pl.BlockSpec((1, d), full)] * 2,
        out_specs=pl.BlockSpec((_BN, 256), blk),
        out_shape=jax.ShapeDtypeStruct((N, 256), jnp.float32),
    )(h, ah0, ah1, at0, at1, Ah, At, bh, bt)


def _tc_head(r0, r1, r2, W1, b1, W2, b2, W3, b3):
    def body(r0_r, r1_r, r2_r, w1_r, b1_r, w2_r, b2_r, w3_r, b3_r, o_r):
        r = r0_r[...] + r1_r[...] + r2_r[...]
        a = jnp.maximum(_mm(r, w1_r[...]) + b1_r[...], 0.0)
        a = jnp.maximum(_mm(a, w2_r[...]) + b2_r[...], 0.0)
        lg = _mm(a, w3_r[...]) + b3_r[...]
        m = jnp.max(lg, axis=1, keepdims=True)
        ex = jnp.exp(lg - m)
        o_r[...] = lg - m - jnp.log(jnp.sum(ex, axis=1, keepdims=True))

    return pl.pallas_call(
        body,
        out_shape=jax.ShapeDtypeStruct((NG, 10), jnp.float32),
    )(r0, r1, r2, W1, b1, W2, b2, W3, b3)


def _antisym(W, d):
    return W - W.T - GAMMA * jnp.eye(d, dtype=jnp.float32)


def kernel(x, edge_index, batch, hom_mask, het_mask, last_epoch,
           pre_W, pre_b,
           hom_W0, hom_b0, hom_P0, het_W0, het_b0, het_P0,
           hom_W1, hom_b1, hom_P1, het_W1, het_b1, het_P1,
           hom_W2, hom_b2, hom_P2, het_W2, het_b2, het_P2,
           lin1_W, lin1_b, lin2_W, lin2_b, lin3_W, lin3_b):
    src = edge_index[0]
    dst = edge_index[1]
    src3 = src.reshape(NTILE, NCHUNK, KC)
    dsth3 = jnp.where(hom_mask, dst, TRASH).reshape(NTILE, NCHUNK, KC)
    dstt3 = jnp.where(het_mask, dst, TRASH).reshape(NTILE, NCHUNK, KC)
    starts = jnp.searchsorted(
        batch, jnp.arange(NG + 1, dtype=jnp.int32)).astype(jnp.int32)
    startsp = jnp.concatenate(
        [starts, jnp.full((72 - (NG + 1),), N, jnp.int32)])

    conv64 = _sc_conv(64)
    conv128 = _sc_conv(128)
    readout = _sc_readout()

    h, yh0, yh1, yt0, yt1 = _tc_a0(x, pre_W, pre_b.reshape(1, -1),
                                   hom_P0, het_P0)
    ah0, ah1, at0, at1 = conv64(src3, dsth3, dstt3, yh0, yh1, yt0, yt1)
    h = _tc_b(h, ah0, ah1, at0, at1,
              _antisym(hom_W0, 128), _antisym(het_W0, 128),
              hom_b0.reshape(1, -1), het_b0.reshape(1, -1), 128, True)
    r0 = readout(h, startsp)

    rs = [r0]
    for (hW, hb, hP), (tW, tb, tP) in (
            ((hom_W1, hom_b1, hom_P1), (het_W1, het_b1, het_P1)),
            ((hom_W2, hom_b2, hom_P2), (het_W2, het_b2, het_P2))):
        yh0, yh1, yt0, yt1 = _tc_a(h, hP, tP)
        ah0, ah1, at0, at1 = conv128(src3, dsth3, dstt3, yh0, yh1, yt0, yt1)
        h = _tc_b(h, ah0, ah1, at0, at1,
                  _antisym(hW, 256), _antisym(tW, 256),
                  hb.reshape(1, -1), tb.reshape(1, -1), 256, False)
        rs.append(readout(h, startsp))

    return _tc_head(rs[0], rs[1], rs[2],
                    lin1_W, lin1_b.reshape(1, -1),
                    lin2_W, lin2_b.reshape(1, -1),
                    lin3_W, lin3_b.reshape(1, -1))


# speedup vs baseline: 2.7897x; 2.7897x over previous
import functools

import jax
import jax.numpy as jnp
from jax import lax
from jax.experimental import pallas as pl
from jax.experimental.pallas import tpu as pltpu
from jax.experimental.pallas import tpu_sc as plsc

N = 10000
E = 320000
NG = 64
EPS = 0.1
GAMMA = 0.1

NTILE = 16
NP = 10240
TRASH = 10000
KC = 80
NROW = E // KC
NCHUNK = NROW // NTILE
NCHUNK2 = NROW // 32
RPT = NP // NTILE
ZR = 64


def _sc_conv(split_edges):
    mesh = plsc.VectorSubcoreMesh(core_axis_name="c", subcore_axis_name="s")
    nchunk = NCHUNK2 if split_edges else NCHUNK
    out = jax.ShapeDtypeStruct((2 * NP, 128), jnp.float32)

    @functools.partial(
        pl.kernel,
        out_type=[out] * 2,
        mesh=mesh,
        scratch_types=[
            pltpu.VMEM((KC,), jnp.int32),
            pltpu.VMEM((KC,), jnp.int32),
            pltpu.VMEM((KC, 128), jnp.float32),
            pltpu.VMEM((ZR, 128), jnp.float32),
            pltpu.VMEM_SHARED((NP, 128), jnp.float32),
            pltpu.SemaphoreType.DMA,
        ],
    )
    def conv(src2, dsth2, dstt2, yh, yt, oh, ot,
             src_c, dst_c, rows_v, zbuf, shared, sem):
        c = lax.axis_index("c")
        s = lax.axis_index("s")
        w = s * 2 + c
        base = (w if split_edges else s) * nchunk
        z16 = jnp.zeros((16,), jnp.float32)

        def zb(j, carry):
            for k in range(8):
                zbuf[j, pl.ds(16 * k, 16)] = z16
            return carry

        lax.fori_loop(0, ZR, zb, 0)

        def run_half(dst2, y, o):
            def zr(j, carry):
                pltpu.sync_copy(zbuf, shared.at[pl.ds(s * RPT + j * ZR, ZR)])
                return carry

            lax.fori_loop(0, RPT // ZR, zr, 0)
            plsc.subcore_barrier()

            def body(i, carry):
                pltpu.sync_copy(src2.at[base + i], src_c)
                pltpu.sync_copy(dst2.at[base + i], dst_c)
                if not split_edges:
                    off = c * N
                    for k in range(KC // 16):
                        src_c[pl.ds(16 * k, 16)] = (
                            src_c[pl.ds(16 * k, 16)] + off)
                pltpu.async_copy(y.at[src_c], rows_v, sem).wait()
                pltpu.sync_copy(rows_v, shared.at[dst_c], add=True)
                return carry

            lax.fori_loop(0, nchunk, body, 0)
            plsc.subcore_barrier()
            pltpu.sync_copy(shared.at[pl.ds(s * RPT, RPT)],
                            o.at[pl.ds(c * NP + s * RPT, RPT)])
            plsc.subcore_barrier()

        run_half(dsth2, yh, oh)
        run_half(dstt2, yt, ot)

    return conv


def _sc_readout():
    mesh = plsc.VectorSubcoreMesh(core_axis_name="c", subcore_axis_name="s")

    @functools.partial(
        pl.kernel,
        out_type=jax.ShapeDtypeStruct((8 * NG, 512), jnp.float32),
        mesh=mesh,
        scratch_types=[
            pltpu.VMEM((80,), jnp.int32),
            pltpu.VMEM((8, 256), jnp.float32),
            pltpu.VMEM((8, 512), jnp.float32),
        ],
    )
    def ro(h_hbm, starts_hbm, out_hbm, starts_v, buf, obuf):
        c = lax.axis_index("c")
        s = lax.axis_index("s")
        wid = s * 2 + c
        pltpu.sync_copy(starts_hbm, starts_v)
        neg = jnp.float32(-3e38)

        for gg in range(2):
            g = wid * 2 + gg
            sv = starts_v[pl.ds(g, 16)]
            rs = sv[0]
            re = sv[1]
            n = re - rs
            a0 = (rs // 8) * 8
            nch = (re - a0 + 7) // 8
            init = (tuple(jnp.zeros((16,), jnp.float32) for _ in range(16)),
                    tuple(jnp.full((16,), neg) for _ in range(16)))

            def chunk(ci, carry, rs=rs, re=re, a0=a0):
                sa, ma = carry
                st = pl.multiple_of(a0 + ci * 8, 8)
                pltpu.sync_copy(h_hbm.at[pl.ds(st, 8)], buf)
                sa = list(sa)
                ma = list(ma)
                for r in range(8):
                    rid = st + r
                    valid = jnp.logical_and(rid >= rs, rid < re)
                    vb = valid.astype(jnp.float32)
                    nvb = (jnp.float32(1.0) - vb) * neg
                    for k in range(16):
                        v = buf[r, pl.ds(16 * k, 16)]
                        sa[k] = sa[k] + v * vb
                        ma[k] = jnp.maximum(ma[k], v * vb + nvb)
                return (tuple(sa), tuple(ma))

            sa, ma = lax.fori_loop(0, nch, chunk, init)
            nf = jnp.zeros((16,), jnp.float32) + n.astype(jnp.float32)
            inv = jnp.ones((16,), jnp.float32) / jnp.maximum(nf, 1.0)
            pos = (n > 0).astype(jnp.float32)
            for k in range(16):
                obuf[0, pl.ds(16 * k, 16)] = ma[k] * pos
                obuf[0, pl.ds(256 + 16 * k, 16)] = sa[k] * inv
            pltpu.sync_copy(obuf, out_hbm.at[pl.ds(pl.multiple_of(g * 8, 8), 8)])

    return ro


_BN = 400
_GN = N // _BN
_BB = 80
_GB = N // _BB


def _mm(a, b):
    return jnp.dot(a, b, preferred_element_type=jnp.float32)


def _tc_a0(x, pre_W, pre_b, Ph, Pt):
    def body(x_r, w_r, b_r, ph_r, pt_r, h_r, yh_r, yt_r):
        h = jnp.maximum(_mm(x_r[...], w_r[...]) + b_r[...], 0.0)
        h_r[...] = h
        yh_r[...] = _mm(h, ph_r[...])
        yt_r[...] = _mm(h, pt_r[...])

    full = lambda i: (0, 0)
    blk = lambda i: (i, 0)
    return pl.pallas_call(
        body,
        grid=(_GN,),
        in_specs=[pl.BlockSpec((_BN, 128), blk),
                  pl.BlockSpec((128, 128), full),
                  pl.BlockSpec((1, 128), full),
                  pl.BlockSpec((128, 128), full),
                  pl.BlockSpec((128, 128), full)],
        out_specs=[pl.BlockSpec((_BN, 128), blk)] * 3,
        out_shape=[jax.ShapeDtypeStruct((N, 128), jnp.float32)] * 3,
    )(x, pre_W, pre_b, Ph, Pt)


def _tc_a(h, Ph, Pt):
    def body(h_r, ph_r, pt_r, yh_r, yt_r):
        hh = h_r[...]
        yh_r[...] = _mm(hh, ph_r[...])
        yt_r[...] = _mm(hh, pt_r[...])

    return pl.pallas_call(
        body,
        grid=(2, _GN),
        in_specs=[pl.BlockSpec((_BN, 256), lambda j, i: (i, 0)),
                  pl.BlockSpec((256, 128), lambda j, i: (0, j)),
                  pl.BlockSpec((256, 128), lambda j, i: (0, j))],
        out_specs=[pl.BlockSpec((_BN, 128),
                                lambda j, i: (j * _GN + i, 0))] * 2,
        out_shape=[jax.ShapeDtypeStruct((2 * N, 128), jnp.float32)] * 2,
    )(h, Ph, Pt)


def _tc_b(h, aggh2, aggt2, Ah, At, bh, bt, d, layer0):
    def body(h_r, a1_r, a2_r, a3_r, a4_r, Ah_r, At_r, bh_r, bt_r, o_r):
        hh = h_r[...]
        if layer0:
            aggh = a1_r[...] + a2_r[...]
            aggt = a3_r[...] + a4_r[...]
        else:
            aggh = jnp.concatenate([a1_r[...], a2_r[...]], axis=1)
            aggt = jnp.concatenate([a3_r[...], a4_r[...]], axis=1)
        xh = jnp.maximum(
            hh + EPS * jnp.tanh(_mm(hh, Ah_r[...]) + aggh + bh_r[...]), 0.0)
        xt = jnp.maximum(
            hh + EPS * jnp.tanh(_mm(hh, At_r[...]) + aggt + bt_r[...]), 0.0)
        if layer0:
            o_r[...] = jnp.concatenate([xh, xt], axis=1)
        else:
            o_r[...] = xh + xt

    full = lambda i: (0, 0)
    blk = lambda i: (i, 0)
    off = NP // _BB
    return pl.pallas_call(
        body,
        grid=(_GB,),
        in_specs=[pl.BlockSpec((_BB, d), blk),
                  pl.BlockSpec((_BB, 128), blk),
                  pl.BlockSpec((_BB, 128), lambda i: (i + off, 0)),
                  pl.BlockSpec((_BB, 128), blk),
                  pl.BlockSpec((_BB, 128), lambda i: (i + off, 0)),
                  pl.BlockSpec((d, d), full),
                  pl.BlockSpec((d, d), full),
                  pl.BlockSpec((1, d), full),
                  pl.BlockSpec((1, d), full)],
        out_specs=pl.BlockSpec((_BB, 256), blk),
        out_shape=jax.ShapeDtypeStruct((N, 256), jnp.float32),
    )(h, aggh2, aggh2, aggt2, aggt2, Ah, At, bh, bt)


def _tc_head(r0, r1, r2, W1, b1, W2, b2, W3, b3):
    def body(r0_r, r1_r, r2_r, w1_r, b1_r, w2_r, b2_r, w3_r, b3_r, o_r):
        r = r0_r[...] + r1_r[...] + r2_r[...]
        a = jnp.maximum(_mm(r, w1_r[...]) + b1_r[...], 0.0)
        a = jnp.maximum(_mm(a, w2_r[...]) + b2_r[...], 0.0)
        lg = _mm(a, w3_r[...]) + b3_r[...]
        m = jnp.max(lg, axis=1, keepdims=True)
        ex = jnp.exp(lg - m)
        o_r[...] = lg - m - jnp.log(jnp.sum(ex, axis=1, keepdims=True))

    return pl.pallas_call(
        body,
        out_shape=jax.ShapeDtypeStruct((NG, 10), jnp.float32),
    )(r0, r1, r2, W1, b1, W2, b2, W3, b3)


def _antisym(W, d):
    return W - W.T - GAMMA * jnp.eye(d, dtype=jnp.float32)


def kernel(x, edge_index, batch, hom_mask, het_mask, last_epoch,
           pre_W, pre_b,
           hom_W0, hom_b0, hom_P0, het_W0, het_b0, het_P0,
           hom_W1, hom_b1, hom_P1, het_W1, het_b1, het_P1,
           hom_W2, hom_b2, hom_P2, het_W2, het_b2, het_P2,
           lin1_W, lin1_b, lin2_W, lin2_b, lin3_W, lin3_b):
    src = edge_index[0]
    dst = edge_index[1]
    src2 = src.reshape(NROW, KC)
    dsth2 = jnp.where(hom_mask, dst, TRASH).reshape(NROW, KC)
    dstt2 = jnp.where(het_mask, dst, TRASH).reshape(NROW, KC)
    starts = jnp.searchsorted(
        batch, jnp.arange(NG + 1, dtype=jnp.int32)).astype(jnp.int32)
    startsp = jnp.concatenate(
        [starts, jnp.full((80 - (NG + 1),), N, jnp.int32)])

    conv_se = _sc_conv(True)
    conv_fs = _sc_conv(False)
    readout = _sc_readout()

    h, yh, yt = _tc_a0(x, pre_W, pre_b.reshape(1, -1), hom_P0, het_P0)
    aggh2, aggt2 = conv_se(src2, dsth2, dstt2, yh, yt)
    h = _tc_b(h, aggh2, aggt2,
              _antisym(hom_W0, 128), _antisym(het_W0, 128),
              hom_b0.reshape(1, -1), het_b0.reshape(1, -1), 128, True)
    r0 = readout(h, startsp)

    rs = [r0]
    for (hW, hb, hP), (tW, tb, tP) in (
            ((hom_W1, hom_b1, hom_P1), (het_W1, het_b1, het_P1)),
            ((hom_W2, hom_b2, hom_P2), (het_W2, het_b2, het_P2))):
        yh2, yt2 = _tc_a(h, hP, tP)
        aggh2, aggt2 = conv_fs(src2, dsth2, dstt2, yh2, yt2)
        h = _tc_b(h, aggh2, aggt2,
                  _antisym(hW, 256), _antisym(tW, 256),
                  hb.reshape(1, -1), tb.reshape(1, -1), 256, False)
        rs.append(readout(h, startsp))

    rs = [r.reshape(NG, 8, 512)[:, 0, :] for r in rs]
    return _tc_head(rs[0], rs[1], rs[2],
                    lin1_W, lin1_b.reshape(1, -1),
                    lin2_W, lin2_b.reshape(1, -1),
                    lin3_W, lin3_b.reshape(1, -1))


# speedup vs baseline: 5.3068x; 1.9023x over previous
import functools

import jax
import jax.numpy as jnp
from jax import lax
from jax.experimental import pallas as pl
from jax.experimental.pallas import tpu as pltpu
from jax.experimental.pallas import tpu_sc as plsc

N = 10000
E = 320000
NG = 64
EPS = 0.1
GAMMA = 0.1

NTILE = 16
NP = 10240
TRASH = 10000
KC = 40
B = 5
NROW = E // KC
NCH_FS = (E // NTILE) // KC
NCH_SE = (E // 32) // KC
RPT = NP // NTILE
ZR = 16


def _sc_conv(split_edges):
    mesh = plsc.VectorSubcoreMesh(core_axis_name="c", subcore_axis_name="s")
    nch = NCH_SE if split_edges else NCH_FS
    ngrp = nch // B
    assert ngrp % 2 == 0
    out = jax.ShapeDtypeStruct((2 * NP, 128), jnp.float32)

    @functools.partial(
        pl.kernel,
        out_type=[out] * 2,
        mesh=mesh,
        scratch_types=[
            pltpu.VMEM((B, KC), jnp.int32),
            pltpu.VMEM((B, KC), jnp.int32),
            pltpu.VMEM((B, KC), jnp.int32),
            pltpu.VMEM((B, KC), jnp.int32),
            pltpu.VMEM((B, KC, 128), jnp.float32),
            pltpu.VMEM((ZR, 128), jnp.float32),
            pltpu.VMEM_SHARED((NP, 128), jnp.float32),
            pltpu.SemaphoreType.DMA,
            pltpu.SemaphoreType.DMA,
            pltpu.SemaphoreType.DMA,
            pltpu.SemaphoreType.DMA,
            pltpu.SemaphoreType.DMA,
            pltpu.SemaphoreType.DMA,
            pltpu.SemaphoreType.DMA,
        ],
    )
    def conv(srcst, dsth2, dstt2, yh, yt, oh, ot,
             is0, id0, is1, id1, rows, zbuf, shared,
             semi, sems, sg0, sg1, sg2, sg3, sg4):
        sgs = (sg0, sg1, sg2, sg3, sg4)
        c = lax.axis_index("c")
        s = lax.axis_index("s")
        w = s * 2 + c
        if split_edges:
            sbase = w * NCH_SE
            dbase = w * NCH_SE
        else:
            sbase = c * NROW + s * NCH_FS
            dbase = s * NCH_FS
        z16 = jnp.zeros((16,), jnp.float32)
        for j in range(ZR):
            for k in range(8):
                zbuf[j, pl.ds(16 * k, 16)] = z16

        def run_half(dst2, y, o):
            for j in range(RPT // ZR):
                pltpu.async_copy(
                    zbuf, shared.at[pl.ds(s * RPT + j * ZR, ZR)], sems)
            for j in range(RPT // ZR):
                pltpu.make_async_copy(
                    zbuf, shared.at[pl.ds(s * RPT, ZR)], sems).wait()
            plsc.subcore_barrier()

            for b in range(B):
                pltpu.async_copy(srcst.at[sbase + b], is0.at[b], semi)
                pltpu.async_copy(dst2.at[dbase + b], id0.at[b], semi)

            def do_group(g, cs, cd, ns, nd):
                for b in range(B):
                    pltpu.make_async_copy(
                        srcst.at[0], cs.at[b], semi).wait()
                    pltpu.make_async_copy(
                        dst2.at[0], cd.at[b], semi).wait()
                snr = jnp.minimum(sbase + (g + 1) * B, 2 * NROW - B)
                dnr = jnp.minimum(dbase + (g + 1) * B, NROW - B)
                for b in range(B):
                    pltpu.async_copy(srcst.at[snr + b], ns.at[b], semi)
                    pltpu.async_copy(dst2.at[dnr + b], nd.at[b], semi)
                for b in range(B):
                    pltpu.async_copy(y.at[cs.at[b]], rows.at[b], sgs[b])
                for b in range(B):
                    pltpu.make_async_copy(
                        y.at[pl.ds(0, KC)], rows.at[b], sgs[b]).wait()
                    pltpu.async_copy(
                        rows.at[b], shared.at[cd.at[b]], sems, add=True)
                for b in range(B):
                    pltpu.make_async_copy(
                        y.at[pl.ds(0, KC)], rows.at[b], sems).wait()

            def gbody(g2, carry):
                do_group(2 * g2, is0, id0, is1, id1)
                do_group(2 * g2 + 1, is1, id1, is0, id0)
                return carry

            lax.fori_loop(0, ngrp // 2, gbody, 0)
            for b in range(B):
                pltpu.make_async_copy(srcst.at[0], is0.at[b], semi).wait()
                pltpu.make_async_copy(dst2.at[0], id0.at[b], semi).wait()
            plsc.subcore_barrier()
            pltpu.sync_copy(shared.at[pl.ds(s * RPT, RPT)],
                            o.at[pl.ds(c * NP + s * RPT, RPT)])
            plsc.subcore_barrier()

        run_half(dsth2, yh, oh)
        run_half(dstt2, yt, ot)

    return conv


def _sc_readout():
    mesh = plsc.VectorSubcoreMesh(core_axis_name="c", subcore_axis_name="s")

    @functools.partial(
        pl.kernel,
        out_type=jax.ShapeDtypeStruct((8 * NG, 512), jnp.float32),
        mesh=mesh,
        scratch_types=[
            pltpu.VMEM((80,), jnp.int32),
            pltpu.VMEM((8, 256), jnp.float32),
            pltpu.VMEM((8, 512), jnp.float32),
        ],
    )
    def ro(h_hbm, starts_hbm, out_hbm, starts_v, buf, obuf):
        c = lax.axis_index("c")
        s = lax.axis_index("s")
        wid = s * 2 + c
        pltpu.sync_copy(starts_hbm, starts_v)
        neg = jnp.float32(-3e38)

        for gg in range(2):
            g = wid * 2 + gg
            sv = starts_v[pl.ds(g, 16)]
            rs = sv[0]
            re = sv[1]
            n = re - rs
            a0 = (rs // 8) * 8
            nch = (re - a0 + 7) // 8
            init = (tuple(jnp.zeros((16,), jnp.float32) for _ in range(16)),
                    tuple(jnp.full((16,), neg) for _ in range(16)))

            def chunk(ci, carry, rs=rs, re=re, a0=a0):
                sa, ma = carry
                st = pl.multiple_of(a0 + ci * 8, 8)
                pltpu.sync_copy(h_hbm.at[pl.ds(st, 8)], buf)
                sa = list(sa)
                ma = list(ma)
                for r in range(8):
                    rid = st + r
                    valid = jnp.logical_and(rid >= rs, rid < re)
                    vb = valid.astype(jnp.float32)
                    nvb = (jnp.float32(1.0) - vb) * neg
                    for k in range(16):
                        v = buf[r, pl.ds(16 * k, 16)]
                        sa[k] = sa[k] + v * vb
                        ma[k] = jnp.maximum(ma[k], v * vb + nvb)
                return (tuple(sa), tuple(ma))

            sa, ma = lax.fori_loop(0, nch, chunk, init)
            nf = jnp.zeros((16,), jnp.float32) + n.astype(jnp.float32)
            inv = jnp.ones((16,), jnp.float32) / jnp.maximum(nf, 1.0)
            pos = (n > 0).astype(jnp.float32)
            for k in range(16):
                obuf[0, pl.ds(16 * k, 16)] = ma[k] * pos
                obuf[0, pl.ds(256 + 16 * k, 16)] = sa[k] * inv
            pltpu.sync_copy(obuf, out_hbm.at[pl.ds(pl.multiple_of(g * 8, 8), 8)])

    return ro


_BN = 400
_GN = N // _BN
_BB = 80
_GB = N // _BB


def _mm(a, b):
    return jnp.dot(a, b, preferred_element_type=jnp.float32)


def _tc_a0(x, pre_W, pre_b, Ph, Pt):
    def body(x_r, w_r, b_r, ph_r, pt_r, h_r, yh_r, yt_r):
        h = jnp.maximum(_mm(x_r[...], w_r[...]) + b_r[...], 0.0)
        h_r[...] = h
        yh_r[...] = _mm(h, ph_r[...])
        yt_r[...] = _mm(h, pt_r[...])

    full = lambda i: (0, 0)
    blk = lambda i: (i, 0)
    return pl.pallas_call(
        body,
        grid=(_GN,),
        in_specs=[pl.BlockSpec((_BN, 128), blk),
                  pl.BlockSpec((128, 128), full),
                  pl.BlockSpec((1, 128), full),
                  pl.BlockSpec((128, 128), full),
                  pl.BlockSpec((128, 128), full)],
        out_specs=[pl.BlockSpec((_BN, 128), blk)] * 3,
        out_shape=[jax.ShapeDtypeStruct((N, 128), jnp.float32)] * 3,
    )(x, pre_W, pre_b, Ph, Pt)


def _tc_a(h, Ph, Pt):
    def body(h_r, ph_r, pt_r, yh_r, yt_r):
        hh = h_r[...]
        yh_r[...] = _mm(hh, ph_r[...])
        yt_r[...] = _mm(hh, pt_r[...])

    return pl.pallas_call(
        body,
        grid=(2, _GN),
        in_specs=[pl.BlockSpec((_BN, 256), lambda j, i: (i, 0)),
                  pl.BlockSpec((256, 128), lambda j, i: (0, j)),
                  pl.BlockSpec((256, 128), lambda j, i: (0, j))],
        out_specs=[pl.BlockSpec((_BN, 128),
                                lambda j, i: (j * _GN + i, 0))] * 2,
        out_shape=[jax.ShapeDtypeStruct((2 * N, 128), jnp.float32)] * 2,
    )(h, Ph, Pt)


def _tc_b(h, aggh2, aggt2, Ah, At, bh, bt, d, layer0):
    def body(h_r, a1_r, a2_r, a3_r, a4_r, Ah_r, At_r, bh_r, bt_r, o_r):
        hh = h_r[...]
        if layer0:
            aggh = a1_r[...] + a2_r[...]
            aggt = a3_r[...] + a4_r[...]
        else:
            aggh = jnp.concatenate([a1_r[...], a2_r[...]], axis=1)
            aggt = jnp.concatenate([a3_r[...], a4_r[...]], axis=1)
        xh = jnp.maximum(
            hh + EPS * jnp.tanh(_mm(hh, Ah_r[...]) + aggh + bh_r[...]), 0.0)
        xt = jnp.maximum(
            hh + EPS * jnp.tanh(_mm(hh, At_r[...]) + aggt + bt_r[...]), 0.0)
        if layer0:
            o_r[...] = jnp.concatenate([xh, xt], axis=1)
        else:
            o_r[...] = xh + xt

    full = lambda i: (0, 0)
    blk = lambda i: (i, 0)
    off = NP // _BB
    return pl.pallas_call(
        body,
        grid=(_GB,),
        in_specs=[pl.BlockSpec((_BB, d), blk),
                  pl.BlockSpec((_BB, 128), blk),
                  pl.BlockSpec((_BB, 128), lambda i: (i + off, 0)),
                  pl.BlockSpec((_BB, 128), blk),
                  pl.BlockSpec((_BB, 128), lambda i: (i + off, 0)),
                  pl.BlockSpec((d, d), full),
                  pl.BlockSpec((d, d), full),
                  pl.BlockSpec((1, d), full),
                  pl.BlockSpec((1, d), full)],
        out_specs=pl.BlockSpec((_BB, 256), blk),
        out_shape=jax.ShapeDtypeStruct((N, 256), jnp.float32),
    )(h, aggh2, aggh2, aggt2, aggt2, Ah, At, bh, bt)


def _tc_head(r0, r1, r2, W1, b1, W2, b2, W3, b3):
    def body(r0_r, r1_r, r2_r, w1_r, b1_r, w2_r, b2_r, w3_r, b3_r, o_r):
        r = r0_r[...] + r1_r[...] + r2_r[...]
        a = jnp.maximum(_mm(r, w1_r[...]) + b1_r[...], 0.0)
        a = jnp.maximum(_mm(a, w2_r[...]) + b2_r[...], 0.0)
        lg = _mm(a, w3_r[...]) + b3_r[...]
        m = jnp.max(lg, axis=1, keepdims=True)
        ex = jnp.exp(lg - m)
        o_r[...] = lg - m - jnp.log(jnp.sum(ex, axis=1, keepdims=True))

    return pl.pallas_call(
        body,
        out_shape=jax.ShapeDtypeStruct((NG, 10), jnp.float32),
    )(r0, r1, r2, W1, b1, W2, b2, W3, b3)


def _antisym(W, d):
    return W - W.T - GAMMA * jnp.eye(d, dtype=jnp.float32)


def kernel(x, edge_index, batch, hom_mask, het_mask, last_epoch,
           pre_W, pre_b,
           hom_W0, hom_b0, hom_P0, het_W0, het_b0, het_P0,
           hom_W1, hom_b1, hom_P1, het_W1, het_b1, het_P1,
           hom_W2, hom_b2, hom_P2, het_W2, het_b2, het_P2,
           lin1_W, lin1_b, lin2_W, lin2_b, lin3_W, lin3_b):
    src = edge_index[0]
    dst = edge_index[1]
    srcst = jnp.concatenate([src, src + N]).reshape(2 * NROW, KC)
    dsth2 = jnp.where(hom_mask, dst, TRASH).reshape(NROW, KC)
    dstt2 = jnp.where(het_mask, dst, TRASH).reshape(NROW, KC)
    starts = jnp.searchsorted(
        batch, jnp.arange(NG + 1, dtype=jnp.int32)).astype(jnp.int32)
    startsp = jnp.concatenate(
        [starts, jnp.full((80 - (NG + 1),), N, jnp.int32)])

    conv_se = _sc_conv(True)
    conv_fs = _sc_conv(False)
    readout = _sc_readout()

    h, yh, yt = _tc_a0(x, pre_W, pre_b.reshape(1, -1), hom_P0, het_P0)
    aggh2, aggt2 = conv_se(srcst, dsth2, dstt2, yh, yt)
    h = _tc_b(h, aggh2, aggt2,
              _antisym(hom_W0, 128), _antisym(het_W0, 128),
              hom_b0.reshape(1, -1), het_b0.reshape(1, -1), 128, True)
    r0 = readout(h, startsp)

    rs = [r0]
    for (hW, hb, hP), (tW, tb, tP) in (
            ((hom_W1, hom_b1, hom_P1), (het_W1, het_b1, het_P1)),
            ((hom_W2, hom_b2, hom_P2), (het_W2, het_b2, het_P2))):
        yh2, yt2 = _tc_a(h, hP, tP)
        aggh2, aggt2 = conv_fs(srcst, dsth2, dstt2, yh2, yt2)
        h = _tc_b(h, aggh2, aggt2,
                  _antisym(hW, 256), _antisym(tW, 256),
                  hb.reshape(1, -1), tb.reshape(1, -1), 256, False)
        rs.append(readout(h, startsp))

    rs = [r.reshape(NG, 8, 512)[:, 0, :] for r in rs]
    return _tc_head(rs[0], rs[1], rs[2],
                    lin1_W, lin1_b.reshape(1, -1),
                    lin2_W, lin2_b.reshape(1, -1),
                    lin3_W, lin3_b.reshape(1, -1))
